# Initial kernel scaffold; baseline (speedup 1.0000x reference)
import jax
import jax.numpy as jnp
from jax.experimental import pallas as pl


def kernel(node_ids, edge_ids, node_emb, edge_emb, W_e, b_e, W_v, b_v):
    raise NotImplementedError("write your pallas kernel here")


# speedup vs baseline: 1.7320x; 1.7320x over previous
import jax
import jax.numpy as jnp
from jax import lax
from jax.experimental import pallas as pl
from jax.experimental.pallas import tpu as pltpu
from jax.experimental.pallas import tpu_sc as plsc

NUM_NODES = 100000
NUM_EDGES = 10000
D = 128
NNZ = 320000

NC, NS = 2, 16
B = 128
B1 = 64
PAD_BLKS = 2560
NPAD = PAD_BLKS * B
S1P = NPAD // (NC * NS)
S2P = NPAD // NS
CH = 2048
EBITS = 14
EMASK = (1 << EBITS) - 1
PAD_NID = (1 << 17) - 1
PAD_EID = 10100
PAD_PACK = PAD_NID * (1 << EBITS) + PAD_EID
EPAD = 10240
E_T = 624
E_TAIL = NUM_EDGES - NS * E_T
CHUNK = 5000
NRANGE = NUM_NODES // CHUNK
CPAD = 5120
DUMP = 5100
C_T = 312
C_TAIL = CHUNK - NS * C_T


def _sc_stage1(pk_hbm, node_emb, zero_hbm, one_hbm,
               esum_out, ecnt_out,
               pkc, nrow, erow, rows, ones, acc):
    c = lax.axis_index("c")
    s = lax.axis_index("s")
    w = c * NS + s
    pltpu.sync_copy(one_hbm.at[pl.ds(0, B1)], ones)

    def zero_acc():
        for k in range(5):
            pltpu.sync_copy(zero_hbm, acc.at[pl.ds(s * 640 + k * B, B)])

    def dump_acc(dst):
        pltpu.sync_copy(acc.at[pl.ds(s * E_T, E_T)],
                        dst.at[c, pl.ds(s * E_T, E_T)])

        @pl.when(s == 0)
        def _():
            pltpu.sync_copy(acc.at[pl.ds(NS * E_T, E_TAIL)],
                            dst.at[c, pl.ds(NS * E_T, E_TAIL)])

    def sweep(phase):
        def chunk_loop(cc, _):
            pltpu.sync_copy(pk_hbm.at[pl.ds(w * S1P + cc * CH, CH)], pkc)

            def blk(j, _):
                for k in range(4):
                    pv = pkc[pl.ds(j * B1 + k * 16, 16)]
                    if phase == 0:
                        nrow[pl.ds(k * 16, 16)] = jnp.minimum(pv >> EBITS,
                                                              NUM_NODES - 1)
                    erow[pl.ds(k * 16, 16)] = pv & EMASK
                if phase == 0:
                    pltpu.sync_copy(node_emb.at[nrow], rows)
                    pltpu.sync_copy(rows, acc.at[erow], add=True)
                else:
                    pltpu.sync_copy(ones, acc.at[erow], add=True)
                return 0

            lax.fori_loop(0, CH // B1, blk, 0)
            return 0

        lax.fori_loop(0, S1P // CH, chunk_loop, 0)
        plsc.subcore_barrier()
        dump_acc(esum_out if phase == 0 else ecnt_out)
        plsc.subcore_barrier()

    zero_acc()
    plsc.subcore_barrier()
    sweep(0)
    zero_acc()
    plsc.subcore_barrier()
    sweep(1)


def _lane_gather(x, idx):
    dn = lax.GatherDimensionNumbers(offset_dims=(), collapsed_slice_dims=(0,),
                                    start_index_map=(0,))
    return lax.gather(x, idx[:, None], dn, (1,),
                      mode=lax.GatherScatterMode.PROMISE_IN_BOUNDS)


def _scalar(v):
    return lax.squeeze(lax.slice(v, (0,), (1,)), (0,))


def _sc_stage2(pk_hbm, ectx_hbm, zero_hbm, one_hbm,
               nsum_out, ncnt_out,
               pkc, comp, rows, ones, sidx, gidx, chunk):
    c = lax.axis_index("c")
    s = lax.axis_index("s")
    iota = lax.iota(jnp.int32, 16)
    pltpu.sync_copy(one_hbm, ones)

    def zero_chunk():
        for off, n in ((0, B), (B, B), (2 * B, 64)):
            pltpu.sync_copy(zero_hbm.at[pl.ds(0, n)],
                            chunk.at[pl.ds(s * 320 + off, n)])

    def dump_chunk(dst, base):
        pltpu.sync_copy(chunk.at[pl.ds(s * C_T, C_T)],
                        dst.at[pl.ds(base + s * C_T, C_T)])

        @pl.when(s == 0)
        def _():
            pltpu.sync_copy(chunk.at[pl.ds(NS * C_T, C_TAIL)],
                            dst.at[pl.ds(base + NS * C_T, C_TAIL)])

    def one_pass(p, _):
        base = (NC * p + c) * CHUNK
        zero_chunk()
        plsc.subcore_barrier()

        def chunk_loop(cc, cur):
            pltpu.sync_copy(pk_hbm.at[pl.ds(s * S2P + cc * CH, CH)], pkc)

            def scan_blk(j, cur):
                basev = iota * 0 + base
                pbasev = basev * (1 << EBITS)
                for k in range(8):
                    pv = pkc[pl.ds(j * B + k * 16, 16)]
                    rel = (pv >> EBITS) - basev
                    oob = lax.shift_right_logical(rel | (CHUNK - 1 - rel), 31)
                    csum = 1 - oob
                    for sh in (1, 2, 4, 8):
                        down = _lane_gather(csum, jnp.maximum(iota - sh, 0))
                        csum = csum + jnp.where(iota >= sh, down, 0)
                    perm = iota * 0
                    tgt = iota + 1
                    for sh in (8, 4, 2, 1):
                        cand = perm + sh
                        cv = _lane_gather(csum, cand - 1)
                        perm = perm + sh * lax.shift_right_logical(cv - tgt,
                                                                   31)
                    comp[pl.ds(cur, 16)] = _lane_gather(pv - pbasev, perm)
                    cur = cur + lax.squeeze(lax.slice(csum, (15,), (16,)),
                                            (0,))
                return cur

            return lax.fori_loop(0, CH // B, scan_blk, cur)

        total = lax.fori_loop(0, S2P // CH, chunk_loop, jnp.int32(0))
        nfl = (total + B - 1) >> 7

        def build_idx(f, totalv, need_g):
            for k in range(8):
                fpk = iota + f * B + k * 16
                valid = fpk < totalv
                pv = comp[pl.ds(f * B + k * 16, 16)]
                sidx[pl.ds(k * 16, 16)] = jnp.where(valid, pv >> EBITS, DUMP)
                if need_g:
                    gidx[pl.ds(k * 16, 16)] = jnp.where(valid, pv & EMASK, 0)

        def flush_a(f, _):
            build_idx(f, iota * 0 + total, True)
            pltpu.sync_copy(ectx_hbm.at[gidx], rows)
            pltpu.sync_copy(rows, chunk.at[sidx], add=True)
            return 0

        def flush_b(f, _):
            build_idx(f, iota * 0 + total, False)
            pltpu.sync_copy(ones, chunk.at[sidx], add=True)
            return 0

        lax.fori_loop(0, nfl, flush_a, 0)
        plsc.subcore_barrier()
        dump_chunk(nsum_out, base)
        plsc.subcore_barrier()
        zero_chunk()
        plsc.subcore_barrier()
        lax.fori_loop(0, nfl, flush_b, 0)
        plsc.subcore_barrier()
        dump_chunk(ncnt_out, base)
        plsc.subcore_barrier()
        return 0

    lax.fori_loop(0, NRANGE // NC, one_pass, 0)


_MESH = plsc.VectorSubcoreMesh(core_axis_name="c", subcore_axis_name="s",
                               num_cores=NC, num_subcores=NS)

_stage1 = pl.kernel(
    _sc_stage1,
    out_type=(jax.ShapeDtypeStruct((NC, NUM_EDGES, D), jnp.float32),
              jax.ShapeDtypeStruct((NC, NUM_EDGES, D), jnp.float32)),
    mesh=_MESH,
    scratch_types=[
        pltpu.VMEM((CH,), jnp.int32),
        pltpu.VMEM((B1,), jnp.int32),
        pltpu.VMEM((B1,), jnp.int32),
        pltpu.VMEM((B1, D), jnp.float32),
        pltpu.VMEM((B1, D), jnp.float32),
        pltpu.VMEM_SHARED((EPAD, D), jnp.float32),
    ],
)

_stage2 = pl.kernel(
    _sc_stage2,
    out_type=(jax.ShapeDtypeStruct((NUM_NODES, D), jnp.float32),
              jax.ShapeDtypeStruct((NUM_NODES, D), jnp.float32)),
    mesh=_MESH,
    scratch_types=[
        pltpu.VMEM((CH,), jnp.int32),
        pltpu.VMEM((S2P,), jnp.int32),
        pltpu.VMEM((B, D), jnp.float32),
        pltpu.VMEM((B, D), jnp.float32),
        pltpu.VMEM((B,), jnp.int32),
        pltpu.VMEM((B,), jnp.int32),
        pltpu.VMEM_SHARED((CPAD, D), jnp.float32),
    ],
)

_BE = 1000
_BN = 2000


def _tc_edge(p_ref, c_ref, emb_ref, w_ref, b_ref, ctx_ref, out_ref):
    esum = p_ref[0] + p_ref[1]
    cnt = c_ref[0][:, 0:1] + c_ref[1][:, 0:1]
    mean = esum / jnp.maximum(cnt, 1.0)
    ctx = jnp.where(cnt > 0, mean, emb_ref[...])
    ctx = lax.dot_general(ctx, w_ref[...], (((1,), (1,)), ((), ())),
                          preferred_element_type=jnp.float32)
    ctx = ctx + b_ref[...]
    ctx_ref[...] = ctx
    out_ref[...] = emb_ref[...] + ctx


def _tc_node(sum_ref, cnt_ref, emb_ref, w_ref, b_ref, out_ref):
    total = cnt_ref[:, 0:1] + 1.0
    ctx = lax.dot_general(sum_ref[...] / total, w_ref[...],
                          (((1,), (1,)), ((), ())),
                          preferred_element_type=jnp.float32) + b_ref[...]
    out_ref[...] = emb_ref[...] + ctx


def _edge_tc(esum_p, ecnt_p, edge_emb, W_e, b_e2):
    return pl.pallas_call(
        _tc_edge,
        grid=(NUM_EDGES // _BE,),
        in_specs=[
            pl.BlockSpec((NC, _BE, D), lambda i: (0, i, 0)),
            pl.BlockSpec((NC, _BE, D), lambda i: (0, i, 0)),
            pl.BlockSpec((_BE, D), lambda i: (i, 0)),
            pl.BlockSpec((D, D), lambda i: (0, 0)),
            pl.BlockSpec((1, D), lambda i: (0, 0)),
        ],
        out_specs=[pl.BlockSpec((_BE, D), lambda i: (i, 0)),
                   pl.BlockSpec((_BE, D), lambda i: (i, 0))],
        out_shape=[jax.ShapeDtypeStruct((NUM_EDGES, D), jnp.float32),
                   jax.ShapeDtypeStruct((NUM_EDGES, D), jnp.float32)],
    )(esum_p, ecnt_p, edge_emb, W_e, b_e2)


def _node_tc(nsum, ncnt, node_emb, W_v, b_v2):
    return pl.pallas_call(
        _tc_node,
        grid=(NUM_NODES // _BN,),
        in_specs=[
            pl.BlockSpec((_BN, D), lambda i: (i, 0)),
            pl.BlockSpec((_BN, D), lambda i: (i, 0)),
            pl.BlockSpec((_BN, D), lambda i: (i, 0)),
            pl.BlockSpec((D, D), lambda i: (0, 0)),
            pl.BlockSpec((1, D), lambda i: (0, 0)),
        ],
        out_specs=pl.BlockSpec((_BN, D), lambda i: (i, 0)),
        out_shape=jax.ShapeDtypeStruct((NUM_NODES, D), jnp.float32),
    )(nsum, ncnt, node_emb, W_v, b_v2)


def kernel(node_ids, edge_ids, node_emb, edge_emb, W_e, b_e, W_v, b_v):
    packed = node_ids.astype(jnp.int32) * (1 << EBITS) + edge_ids.astype(
        jnp.int32)
    packed = jnp.pad(packed, (0, NPAD - NNZ), constant_values=PAD_PACK)
    zero_hbm = jnp.zeros((B, D), jnp.float32)
    one_hbm = jnp.ones((B, D), jnp.float32)
    esum_p, ecnt_p = _stage1(packed, node_emb, zero_hbm, one_hbm)
    edge_ctx, edge_out = _edge_tc(esum_p, ecnt_p, edge_emb, W_e,
                                  b_e.reshape(1, D))
    nsum, ncnt = _stage2(packed, edge_ctx, zero_hbm, one_hbm)
    node_out = _node_tc(nsum, ncnt, node_emb, W_v, b_v.reshape(1, D))
    return (node_out, edge_out)


# speedup vs baseline: 1.9997x; 1.1546x over previous
import jax
import jax.numpy as jnp
from jax import lax
from jax.experimental import pallas as pl
from jax.experimental.pallas import tpu as pltpu
from jax.experimental.pallas import tpu_sc as plsc

NUM_NODES = 100000
NUM_EDGES = 10000
D = 128
NNZ = 320000

NC, NS = 2, 16
B = 128
B1 = 64
B2 = 64
PAD_BLKS = 2560
NPAD = PAD_BLKS * B
S1P = NPAD // (NC * NS)
S2P = NPAD // NS
CH = 2048
EBITS = 14
EMASK = (1 << EBITS) - 1
PAD_NID = (1 << 17) - 1
PAD_EID = 10100
PAD_PACK = PAD_NID * (1 << EBITS) + PAD_EID
EPAD = 10240
E_T = 624
E_TAIL = NUM_EDGES - NS * E_T
CHUNK = 5000
NRANGE = NUM_NODES // CHUNK
CPAD = 5120
DUMP = 5100
C_T = 312
C_TAIL = CHUNK - NS * C_T


def _sc_stage1(pk_hbm, node_emb, zero_hbm, one_hbm,
               esum_out, ecnt_out,
               pkc, nrow, erow, rows, ones, acc):
    c = lax.axis_index("c")
    s = lax.axis_index("s")
    w = c * NS + s
    pltpu.sync_copy(one_hbm.at[pl.ds(0, B1)], ones)

    def zero_acc():
        for k in range(5):
            pltpu.sync_copy(zero_hbm, acc.at[pl.ds(s * 640 + k * B, B)])

    def dump_acc(dst):
        pltpu.sync_copy(acc.at[pl.ds(s * E_T, E_T)],
                        dst.at[c, pl.ds(s * E_T, E_T)])

        @pl.when(s == 0)
        def _():
            pltpu.sync_copy(acc.at[pl.ds(NS * E_T, E_TAIL)],
                            dst.at[c, pl.ds(NS * E_T, E_TAIL)])

    def sweep(phase):
        def chunk_loop(cc, _):
            pltpu.sync_copy(pk_hbm.at[pl.ds(w * S1P + cc * CH, CH)], pkc)

            def blk(j, _):
                for k in range(4):
                    pv = pkc[pl.ds(j * B1 + k * 16, 16)]
                    if phase == 0:
                        nrow[pl.ds(k * 16, 16)] = jnp.minimum(pv >> EBITS,
                                                              NUM_NODES - 1)
                    erow[pl.ds(k * 16, 16)] = pv & EMASK
                if phase == 0:
                    pltpu.sync_copy(node_emb.at[nrow], rows)
                    pltpu.sync_copy(rows, acc.at[erow], add=True)
                else:
                    pltpu.sync_copy(ones, acc.at[erow], add=True)
                return 0

            lax.fori_loop(0, CH // B1, blk, 0)
            return 0

        lax.fori_loop(0, S1P // CH, chunk_loop, 0)
        plsc.subcore_barrier()
        dump_acc(esum_out if phase == 0 else ecnt_out)
        plsc.subcore_barrier()

    zero_acc()
    plsc.subcore_barrier()
    sweep(0)
    zero_acc()
    plsc.subcore_barrier()
    sweep(1)


def _lane_gather(x, idx):
    dn = lax.GatherDimensionNumbers(offset_dims=(), collapsed_slice_dims=(0,),
                                    start_index_map=(0,))
    return lax.gather(x, idx[:, None], dn, (1,),
                      mode=lax.GatherScatterMode.PROMISE_IN_BOUNDS)


def _scalar(v):
    return lax.squeeze(lax.slice(v, (0,), (1,)), (0,))


def _sc_stage2(pk_hbm, ectx_hbm, zero_hbm, one_hbm,
               nsum_out, ncnt_out,
               pkc, comp, rows_a, rows_b, ones, sidx, sidx8, gidx, gidx2,
               chunk, sem_a, sem_b):
    c = lax.axis_index("c")
    s = lax.axis_index("s")
    iota = lax.iota(jnp.int32, 16)
    pltpu.sync_copy(one_hbm, ones)

    def zero_chunk():
        for off, n in ((0, B), (B, B), (2 * B, 64)):
            pltpu.sync_copy(zero_hbm.at[pl.ds(0, n)],
                            chunk.at[pl.ds(s * 320 + off, n)])

    def dump_chunk(dst, base):
        pltpu.sync_copy(chunk.at[pl.ds(s * C_T, C_T)],
                        dst.at[pl.ds(base + s * C_T, C_T)])

        @pl.when(s == 0)
        def _():
            pltpu.sync_copy(chunk.at[pl.ds(NS * C_T, C_TAIL)],
                            dst.at[pl.ds(base + NS * C_T, C_TAIL)])

    def one_pass(p, _):
        base = (NC * p + c) * CHUNK
        zero_chunk()
        plsc.subcore_barrier()

        def chunk_loop(cc, cur):
            pltpu.sync_copy(pk_hbm.at[pl.ds(s * S2P + cc * CH, CH)], pkc)

            def scan_blk(j, cur):
                basev = iota * 0 + base
                pbasev = basev * (1 << EBITS)
                for k in range(8):
                    pv = pkc[pl.ds(j * B + k * 16, 16)]
                    rel = (pv >> EBITS) - basev
                    oob = lax.shift_right_logical(rel | (CHUNK - 1 - rel), 31)
                    csum = 1 - oob
                    for sh in (1, 2, 4, 8):
                        down = _lane_gather(csum, jnp.maximum(iota - sh, 0))
                        csum = csum + jnp.where(iota >= sh, down, 0)
                    perm = iota * 0
                    tgt = iota + 1
                    for sh in (8, 4, 2, 1):
                        cand = perm + sh
                        cv = _lane_gather(csum, cand - 1)
                        perm = perm + sh * lax.shift_right_logical(cv - tgt,
                                                                   31)
                    comp[pl.ds(cur, 16)] = _lane_gather(pv - pbasev, perm)
                    cur = cur + lax.squeeze(lax.slice(csum, (15,), (16,)),
                                            (0,))
                return cur

            return lax.fori_loop(0, CH // B, scan_blk, cur)

        total = lax.fori_loop(0, S2P // CH, chunk_loop, jnp.int32(0))
        nfl = (total + B - 1) >> 7
        nf6 = (total + B2 - 1) >> 6

        def build_g(f, g):
            for k in range(4):
                fpk = iota + f * B2 + k * 16
                valid = fpk < iota * 0 + total
                pv = comp[pl.ds(f * B2 + k * 16, 16)]
                g[pl.ds(k * 16, 16)] = jnp.where(valid, pv & EMASK, 0)

        def build_s(f):
            for k in range(4):
                fpk = iota + f * B2 + k * 16
                valid = fpk < iota * 0 + total
                pv = comp[pl.ds(f * B2 + k * 16, 16)]
                sidx[pl.ds(k * 16, 16)] = jnp.where(valid, pv >> EBITS, DUMP)

        def build_idx(f, totalv, need_g):
            for k in range(8):
                fpk = iota + f * B + k * 16
                valid = fpk < totalv
                pv = comp[pl.ds(f * B + k * 16, 16)]
                sidx8[pl.ds(k * 16, 16)] = jnp.where(valid, pv >> EBITS, DUMP)

        def flush_b(f, _):
            build_idx(f, iota * 0 + total, False)
            pltpu.sync_copy(ones, chunk.at[sidx8], add=True)
            return 0

        @pl.when(nf6 > 0)
        def _():
            build_g(0, gidx)
            pltpu.async_copy(ectx_hbm.at[gidx], rows_a, sem_a).wait()

        def flush_pair(f2, _):
            f0 = 2 * f2
            f1 = f0 + 1

            @pl.when(f1 < nf6)
            def _():
                build_g(f1, gidx2)
                pltpu.async_copy(ectx_hbm.at[gidx2], rows_b, sem_b)
            build_s(f0)
            pltpu.sync_copy(rows_a, chunk.at[sidx], add=True)

            @pl.when(f1 < nf6)
            def _():
                pltpu.make_async_copy(ectx_hbm.at[gidx2], rows_b, sem_b).wait()

                @pl.when(f1 + 1 < nf6)
                def _():
                    build_g(f1 + 1, gidx)
                    pltpu.async_copy(ectx_hbm.at[gidx], rows_a, sem_a)
                build_s(f1)
                pltpu.sync_copy(rows_b, chunk.at[sidx], add=True)

                @pl.when(f1 + 1 < nf6)
                def _():
                    pltpu.make_async_copy(ectx_hbm.at[gidx], rows_a,
                                          sem_a).wait()
            return 0

        lax.fori_loop(0, (nf6 + 1) >> 1, flush_pair, 0)
        plsc.subcore_barrier()
        dump_chunk(nsum_out, base)
        plsc.subcore_barrier()
        zero_chunk()
        plsc.subcore_barrier()
        lax.fori_loop(0, nfl, flush_b, 0)
        plsc.subcore_barrier()
        dump_chunk(ncnt_out, base)
        plsc.subcore_barrier()
        return 0

    lax.fori_loop(0, NRANGE // NC, one_pass, 0)


_MESH = plsc.VectorSubcoreMesh(core_axis_name="c", subcore_axis_name="s",
                               num_cores=NC, num_subcores=NS)

_stage1 = pl.kernel(
    _sc_stage1,
    out_type=(jax.ShapeDtypeStruct((NC, NUM_EDGES, D), jnp.float32),
              jax.ShapeDtypeStruct((NC, NUM_EDGES, D), jnp.float32)),
    mesh=_MESH,
    scratch_types=[
        pltpu.VMEM((CH,), jnp.int32),
        pltpu.VMEM((B1,), jnp.int32),
        pltpu.VMEM((B1,), jnp.int32),
        pltpu.VMEM((B1, D), jnp.float32),
        pltpu.VMEM((B1, D), jnp.float32),
        pltpu.VMEM_SHARED((EPAD, D), jnp.float32),
    ],
)

_stage2 = pl.kernel(
    _sc_stage2,
    out_type=(jax.ShapeDtypeStruct((NUM_NODES, D), jnp.float32),
              jax.ShapeDtypeStruct((NUM_NODES, D), jnp.float32)),
    mesh=_MESH,
    scratch_types=[
        pltpu.VMEM((CH,), jnp.int32),
        pltpu.VMEM((S2P,), jnp.int32),
        pltpu.VMEM((B2, D), jnp.float32),
        pltpu.VMEM((B2, D), jnp.float32),
        pltpu.VMEM((B, D), jnp.float32),
        pltpu.VMEM((B2,), jnp.int32),
        pltpu.VMEM((B,), jnp.int32),
        pltpu.VMEM((B2,), jnp.int32),
        pltpu.VMEM((B2,), jnp.int32),
        pltpu.VMEM_SHARED((CPAD, D), jnp.float32),
        pltpu.SemaphoreType.DMA,
        pltpu.SemaphoreType.DMA,
    ],
)

_BE = 1000
_BN = 2000


def _tc_edge(p_ref, c_ref, emb_ref, w_ref, b_ref, ctx_ref, out_ref):
    esum = p_ref[0] + p_ref[1]
    cnt = c_ref[0][:, 0:1] + c_ref[1][:, 0:1]
    mean = esum / jnp.maximum(cnt, 1.0)
    ctx = jnp.where(cnt > 0, mean, emb_ref[...])
    ctx = lax.dot_general(ctx, w_ref[...], (((1,), (1,)), ((), ())),
                          preferred_element_type=jnp.float32)
    ctx = ctx + b_ref[...]
    ctx_ref[...] = ctx
    out_ref[...] = emb_ref[...] + ctx


def _tc_node(sum_ref, cnt_ref, emb_ref, w_ref, b_ref, out_ref):
    total = cnt_ref[:, 0:1] + 1.0
    ctx = lax.dot_general(sum_ref[...] / total, w_ref[...],
                          (((1,), (1,)), ((), ())),
                          preferred_element_type=jnp.float32) + b_ref[...]
    out_ref[...] = emb_ref[...] + ctx


def _edge_tc(esum_p, ecnt_p, edge_emb, W_e, b_e2):
    return pl.pallas_call(
        _tc_edge,
        grid=(NUM_EDGES // _BE,),
        in_specs=[
            pl.BlockSpec((NC, _BE, D), lambda i: (0, i, 0)),
            pl.BlockSpec((NC, _BE, D), lambda i: (0, i, 0)),
            pl.BlockSpec((_BE, D), lambda i: (i, 0)),
            pl.BlockSpec((D, D), lambda i: (0, 0)),
            pl.BlockSpec((1, D), lambda i: (0, 0)),
        ],
        out_specs=[pl.BlockSpec((_BE, D), lambda i: (i, 0)),
                   pl.BlockSpec((_BE, D), lambda i: (i, 0))],
        out_shape=[jax.ShapeDtypeStruct((NUM_EDGES, D), jnp.float32),
                   jax.ShapeDtypeStruct((NUM_EDGES, D), jnp.float32)],
    )(esum_p, ecnt_p, edge_emb, W_e, b_e2)


def _node_tc(nsum, ncnt, node_emb, W_v, b_v2):
    return pl.pallas_call(
        _tc_node,
        grid=(NUM_NODES // _BN,),
        in_specs=[
            pl.BlockSpec((_BN, D), lambda i: (i, 0)),
            pl.BlockSpec((_BN, D), lambda i: (i, 0)),
            pl.BlockSpec((_BN, D), lambda i: (i, 0)),
            pl.BlockSpec((D, D), lambda i: (0, 0)),
            pl.BlockSpec((1, D), lambda i: (0, 0)),
        ],
        out_specs=pl.BlockSpec((_BN, D), lambda i: (i, 0)),
        out_shape=jax.ShapeDtypeStruct((NUM_NODES, D), jnp.float32),
    )(nsum, ncnt, node_emb, W_v, b_v2)


def kernel(node_ids, edge_ids, node_emb, edge_emb, W_e, b_e, W_v, b_v):
    packed = node_ids.astype(jnp.int32) * (1 << EBITS) + edge_ids.astype(
        jnp.int32)
    packed = jnp.pad(packed, (0, NPAD - NNZ), constant_values=PAD_PACK)
    zero_hbm = jnp.zeros((B, D), jnp.float32)
    one_hbm = jnp.ones((B, D), jnp.float32)
    esum_p, ecnt_p = _stage1(packed, node_emb, zero_hbm, one_hbm)
    edge_ctx, edge_out = _edge_tc(esum_p, ecnt_p, edge_emb, W_e,
                                  b_e.reshape(1, D))
    nsum, ncnt = _stage2(packed, edge_ctx, zero_hbm, one_hbm)
    node_out = _node_tc(nsum, ncnt, node_emb, W_v, b_v.reshape(1, D))
    return (node_out, edge_out)


# speedup vs baseline: 2.0720x; 1.0361x over previous
import jax
import jax.numpy as jnp
from jax import lax
from jax.experimental import pallas as pl
from jax.experimental.pallas import tpu as pltpu
from jax.experimental.pallas import tpu_sc as plsc

NUM_NODES = 100000
NUM_EDGES = 10000
D = 128
NNZ = 320000

NC, NS = 2, 16
B = 128
B1 = 64
B2 = 64
PAD_BLKS = 2560
NPAD = PAD_BLKS * B
S1P = NPAD // (NC * NS)
S2P = NPAD // NS
CH = 2048
EBITS = 14
EMASK = (1 << EBITS) - 1
PAD_NID = (1 << 17) - 1
PAD_EID = 10100
PAD_PACK = PAD_NID * (1 << EBITS) + PAD_EID
EPAD = 10240
E_T = 624
E_TAIL = NUM_EDGES - NS * E_T
CHUNK = 5000
NRANGE = NUM_NODES // CHUNK
CPAD = 5120
DUMP = 5100
C_T = 312
C_TAIL = CHUNK - NS * C_T


def _sc_stage1(pk_hbm, node_emb, zero_hbm, one_hbm,
               esum_out, ecnt_out,
               pkc, nrow, erow, nrow2, erow2, rows, rows2, ones, acc,
               sem_a, sem_b):
    c = lax.axis_index("c")
    s = lax.axis_index("s")
    w = c * NS + s
    pltpu.sync_copy(one_hbm.at[pl.ds(0, B1)], ones)

    def zero_acc():
        for k in range(5):
            pltpu.sync_copy(zero_hbm, acc.at[pl.ds(s * 640 + k * B, B)])

    def dump_acc(dst):
        pltpu.sync_copy(acc.at[pl.ds(s * E_T, E_T)],
                        dst.at[c, pl.ds(s * E_T, E_T)])

        @pl.when(s == 0)
        def _():
            pltpu.sync_copy(acc.at[pl.ds(NS * E_T, E_TAIL)],
                            dst.at[c, pl.ds(NS * E_T, E_TAIL)])

    NBLK1 = CH // B1

    def _build(j, nr, er, need_n):
        for k in range(4):
            pv = pkc[pl.ds(j * B1 + k * 16, 16)]
            if need_n:
                nr[pl.ds(k * 16, 16)] = jnp.minimum(pv >> EBITS,
                                                    NUM_NODES - 1)
            er[pl.ds(k * 16, 16)] = pv & EMASK

    def sweep(phase):
        def chunk_loop(cc, _):
            pltpu.sync_copy(pk_hbm.at[pl.ds(w * S1P + cc * CH, CH)], pkc)

            if phase == 1:
                def blk(j, _):
                    _build(j, nrow, erow, False)
                    pltpu.sync_copy(ones, acc.at[erow], add=True)
                    return 0

                lax.fori_loop(0, NBLK1, blk, 0)
            else:
                _build(0, nrow, erow, True)
                pltpu.async_copy(node_emb.at[nrow], rows, sem_a).wait()

                def pair(j2, _):
                    j0 = 2 * j2
                    _build(j0 + 1, nrow2, erow2, True)
                    pltpu.async_copy(node_emb.at[nrow2], rows2, sem_b)
                    pltpu.sync_copy(rows, acc.at[erow], add=True)

                    @pl.when(j2 < NBLK1 // 2 - 1)
                    def _():
                        _build(j0 + 2, nrow, erow, True)
                        pltpu.async_copy(node_emb.at[nrow], rows, sem_a)
                    pltpu.make_async_copy(node_emb.at[nrow2], rows2,
                                          sem_b).wait()
                    pltpu.sync_copy(rows2, acc.at[erow2], add=True)

                    @pl.when(j2 < NBLK1 // 2 - 1)
                    def _():
                        pltpu.make_async_copy(node_emb.at[nrow], rows,
                                              sem_a).wait()
                    return 0

                lax.fori_loop(0, NBLK1 // 2, pair, 0)
            return 0

        lax.fori_loop(0, S1P // CH, chunk_loop, 0)
        plsc.subcore_barrier()
        dump_acc(esum_out if phase == 0 else ecnt_out)
        plsc.subcore_barrier()

    zero_acc()
    plsc.subcore_barrier()
    sweep(0)
    zero_acc()
    plsc.subcore_barrier()
    sweep(1)


def _lane_gather(x, idx):
    dn = lax.GatherDimensionNumbers(offset_dims=(), collapsed_slice_dims=(0,),
                                    start_index_map=(0,))
    return lax.gather(x, idx[:, None], dn, (1,),
                      mode=lax.GatherScatterMode.PROMISE_IN_BOUNDS)


def _scalar(v):
    return lax.squeeze(lax.slice(v, (0,), (1,)), (0,))


def _sc_stage2(pk_hbm, ectx_hbm, zero_hbm, one_hbm,
               nsum_out, ncnt_out,
               pkc, comp, rows_a, rows_b, ones, sidx, sidx8, gidx, gidx2,
               chunk, sem_a, sem_b):
    c = lax.axis_index("c")
    s = lax.axis_index("s")
    iota = lax.iota(jnp.int32, 16)
    pltpu.sync_copy(one_hbm, ones)

    def zero_chunk():
        for off, n in ((0, B), (B, B), (2 * B, 64)):
            pltpu.sync_copy(zero_hbm.at[pl.ds(0, n)],
                            chunk.at[pl.ds(s * 320 + off, n)])

    def dump_chunk(dst, base):
        pltpu.sync_copy(chunk.at[pl.ds(s * C_T, C_T)],
                        dst.at[pl.ds(base + s * C_T, C_T)])

        @pl.when(s == 0)
        def _():
            pltpu.sync_copy(chunk.at[pl.ds(NS * C_T, C_TAIL)],
                            dst.at[pl.ds(base + NS * C_T, C_TAIL)])

    def one_pass(p, _):
        base = (NC * p + c) * CHUNK
        zero_chunk()
        plsc.subcore_barrier()

        def chunk_loop(cc, cur):
            pltpu.sync_copy(pk_hbm.at[pl.ds(s * S2P + cc * CH, CH)], pkc)

            def scan_blk(j, cur):
                basev = iota * 0 + base
                pbasev = basev * (1 << EBITS)
                for k in range(8):
                    pv = pkc[pl.ds(j * B + k * 16, 16)]
                    rel = (pv >> EBITS) - basev
                    oob = lax.shift_right_logical(rel | (CHUNK - 1 - rel), 31)
                    csum = 1 - oob
                    for sh in (1, 2, 4, 8):
                        down = _lane_gather(csum, jnp.maximum(iota - sh, 0))
                        csum = csum + jnp.where(iota >= sh, down, 0)
                    perm = iota * 0
                    tgt = iota + 1
                    for sh in (8, 4, 2, 1):
                        cand = perm + sh
                        cv = _lane_gather(csum, cand - 1)
                        perm = perm + sh * lax.shift_right_logical(cv - tgt,
                                                                   31)
                    comp[pl.ds(cur, 16)] = _lane_gather(pv - pbasev, perm)
                    cur = cur + lax.squeeze(lax.slice(csum, (15,), (16,)),
                                            (0,))
                return cur

            return lax.fori_loop(0, CH // B, scan_blk, cur)

        total = lax.fori_loop(0, S2P // CH, chunk_loop, jnp.int32(0))
        nfl = (total + B - 1) >> 7
        nf6 = (total + B2 - 1) >> 6

        def build_g(f, g):
            for k in range(4):
                fpk = iota + f * B2 + k * 16
                valid = fpk < iota * 0 + total
                pv = comp[pl.ds(f * B2 + k * 16, 16)]
                g[pl.ds(k * 16, 16)] = jnp.where(valid, pv & EMASK, 0)

        def build_s(f):
            for k in range(4):
                fpk = iota + f * B2 + k * 16
                valid = fpk < iota * 0 + total
                pv = comp[pl.ds(f * B2 + k * 16, 16)]
                sidx[pl.ds(k * 16, 16)] = jnp.where(valid, pv >> EBITS, DUMP)

        def build_idx(f, totalv, need_g):
            for k in range(8):
                fpk = iota + f * B + k * 16
                valid = fpk < totalv
                pv = comp[pl.ds(f * B + k * 16, 16)]
                sidx8[pl.ds(k * 16, 16)] = jnp.where(valid, pv >> EBITS, DUMP)

        def flush_b(f, _):
            build_idx(f, iota * 0 + total, False)
            pltpu.sync_copy(ones, chunk.at[sidx8], add=True)
            return 0

        @pl.when(nf6 > 0)
        def _():
            build_g(0, gidx)
            pltpu.async_copy(ectx_hbm.at[gidx], rows_a, sem_a).wait()

        def flush_pair(f2, _):
            f0 = 2 * f2
            f1 = f0 + 1

            @pl.when(f1 < nf6)
            def _():
                build_g(f1, gidx2)
                pltpu.async_copy(ectx_hbm.at[gidx2], rows_b, sem_b)
            build_s(f0)
            pltpu.sync_copy(rows_a, chunk.at[sidx], add=True)

            @pl.when(f1 < nf6)
            def _():
                pltpu.make_async_copy(ectx_hbm.at[gidx2], rows_b, sem_b).wait()

                @pl.when(f1 + 1 < nf6)
                def _():
                    build_g(f1 + 1, gidx)
                    pltpu.async_copy(ectx_hbm.at[gidx], rows_a, sem_a)
                build_s(f1)
                pltpu.sync_copy(rows_b, chunk.at[sidx], add=True)

                @pl.when(f1 + 1 < nf6)
                def _():
                    pltpu.make_async_copy(ectx_hbm.at[gidx], rows_a,
                                          sem_a).wait()
            return 0

        lax.fori_loop(0, (nf6 + 1) >> 1, flush_pair, 0)
        plsc.subcore_barrier()
        dump_chunk(nsum_out, base)
        plsc.subcore_barrier()
        zero_chunk()
        plsc.subcore_barrier()
        lax.fori_loop(0, nfl, flush_b, 0)
        plsc.subcore_barrier()
        dump_chunk(ncnt_out, base)
        plsc.subcore_barrier()
        return 0

    lax.fori_loop(0, NRANGE // NC, one_pass, 0)


_MESH = plsc.VectorSubcoreMesh(core_axis_name="c", subcore_axis_name="s",
                               num_cores=NC, num_subcores=NS)

_stage1 = pl.kernel(
    _sc_stage1,
    out_type=(jax.ShapeDtypeStruct((NC, NUM_EDGES, D), jnp.float32),
              jax.ShapeDtypeStruct((NC, NUM_EDGES, D), jnp.float32)),
    mesh=_MESH,
    scratch_types=[
        pltpu.VMEM((CH,), jnp.int32),
        pltpu.VMEM((B1,), jnp.int32),
        pltpu.VMEM((B1,), jnp.int32),
        pltpu.VMEM((B1,), jnp.int32),
        pltpu.VMEM((B1,), jnp.int32),
        pltpu.VMEM((B1, D), jnp.float32),
        pltpu.VMEM((B1, D), jnp.float32),
        pltpu.VMEM((B1, D), jnp.float32),
        pltpu.VMEM_SHARED((EPAD, D), jnp.float32),
        pltpu.SemaphoreType.DMA,
        pltpu.SemaphoreType.DMA,
    ],
)

_stage2 = pl.kernel(
    _sc_stage2,
    out_type=(jax.ShapeDtypeStruct((NUM_NODES, D), jnp.float32),
              jax.ShapeDtypeStruct((NUM_NODES, D), jnp.float32)),
    mesh=_MESH,
    scratch_types=[
        pltpu.VMEM((CH,), jnp.int32),
        pltpu.VMEM((S2P,), jnp.int32),
        pltpu.VMEM((B2, D), jnp.float32),
        pltpu.VMEM((B2, D), jnp.float32),
        pltpu.VMEM((B, D), jnp.float32),
        pltpu.VMEM((B2,), jnp.int32),
        pltpu.VMEM((B,), jnp.int32),
        pltpu.VMEM((B2,), jnp.int32),
        pltpu.VMEM((B2,), jnp.int32),
        pltpu.VMEM_SHARED((CPAD, D), jnp.float32),
        pltpu.SemaphoreType.DMA,
        pltpu.SemaphoreType.DMA,
    ],
)

_BE = 1000
_BN = 2000


def _tc_edge(p_ref, c_ref, emb_ref, w_ref, b_ref, ctx_ref, out_ref):
    esum = p_ref[0] + p_ref[1]
    cnt = c_ref[0][:, 0:1] + c_ref[1][:, 0:1]
    mean = esum / jnp.maximum(cnt, 1.0)
    ctx = jnp.where(cnt > 0, mean, emb_ref[...])
    ctx = lax.dot_general(ctx, w_ref[...], (((1,), (1,)), ((), ())),
                          preferred_element_type=jnp.float32)
    ctx = ctx + b_ref[...]
    ctx_ref[...] = ctx
    out_ref[...] = emb_ref[...] + ctx


def _tc_node(sum_ref, cnt_ref, emb_ref, w_ref, b_ref, out_ref):
    total = cnt_ref[:, 0:1] + 1.0
    ctx = lax.dot_general(sum_ref[...] / total, w_ref[...],
                          (((1,), (1,)), ((), ())),
                          preferred_element_type=jnp.float32) + b_ref[...]
    out_ref[...] = emb_ref[...] + ctx


def _edge_tc(esum_p, ecnt_p, edge_emb, W_e, b_e2):
    return pl.pallas_call(
        _tc_edge,
        grid=(NUM_EDGES // _BE,),
        in_specs=[
            pl.BlockSpec((NC, _BE, D), lambda i: (0, i, 0)),
            pl.BlockSpec((NC, _BE, D), lambda i: (0, i, 0)),
            pl.BlockSpec((_BE, D), lambda i: (i, 0)),
            pl.BlockSpec((D, D), lambda i: (0, 0)),
            pl.BlockSpec((1, D), lambda i: (0, 0)),
        ],
        out_specs=[pl.BlockSpec((_BE, D), lambda i: (i, 0)),
                   pl.BlockSpec((_BE, D), lambda i: (i, 0))],
        out_shape=[jax.ShapeDtypeStruct((NUM_EDGES, D), jnp.float32),
                   jax.ShapeDtypeStruct((NUM_EDGES, D), jnp.float32)],
    )(esum_p, ecnt_p, edge_emb, W_e, b_e2)


def _node_tc(nsum, ncnt, node_emb, W_v, b_v2):
    return pl.pallas_call(
        _tc_node,
        grid=(NUM_NODES // _BN,),
        in_specs=[
            pl.BlockSpec((_BN, D), lambda i: (i, 0)),
            pl.BlockSpec((_BN, D), lambda i: (i, 0)),
            pl.BlockSpec((_BN, D), lambda i: (i, 0)),
            pl.BlockSpec((D, D), lambda i: (0, 0)),
            pl.BlockSpec((1, D), lambda i: (0, 0)),
        ],
        out_specs=pl.BlockSpec((_BN, D), lambda i: (i, 0)),
        out_shape=jax.ShapeDtypeStruct((NUM_NODES, D), jnp.float32),
    )(nsum, ncnt, node_emb, W_v, b_v2)


def kernel(node_ids, edge_ids, node_emb, edge_emb, W_e, b_e, W_v, b_v):
    packed = node_ids.astype(jnp.int32) * (1 << EBITS) + edge_ids.astype(
        jnp.int32)
    packed = jnp.pad(packed, (0, NPAD - NNZ), constant_values=PAD_PACK)
    zero_hbm = jnp.zeros((B, D), jnp.float32)
    one_hbm = jnp.ones((B, D), jnp.float32)
    esum_p, ecnt_p = _stage1(packed, node_emb, zero_hbm, one_hbm)
    edge_ctx, edge_out = _edge_tc(esum_p, ecnt_p, edge_emb, W_e,
                                  b_e.reshape(1, D))
    nsum, ncnt = _stage2(packed, edge_ctx, zero_hbm, one_hbm)
    node_out = _node_tc(nsum, ncnt, node_emb, W_v, b_v.reshape(1, D))
    return (node_out, edge_out)


# speedup vs baseline: 2.1080x; 1.0174x over previous
import jax
import jax.numpy as jnp
from jax import lax
from jax.experimental import pallas as pl
from jax.experimental.pallas import tpu as pltpu
from jax.experimental.pallas import tpu_sc as plsc

NUM_NODES = 100000
NUM_EDGES = 10000
D = 128
NNZ = 320000

NC, NS = 2, 16
B = 128
B1 = 64
B2 = 64
PAD_BLKS = 2560
NPAD = PAD_BLKS * B
S1P = NPAD // (NC * NS)
S2P = NPAD // NS
CH = 2048
EBITS = 14
EMASK = (1 << EBITS) - 1
PAD_NID = (1 << 17) - 1
PAD_EID = 10100
PAD_PACK = PAD_NID * (1 << EBITS) + PAD_EID
EPAD = 10240
E_T = 624
E_TAIL = NUM_EDGES - NS * E_T
CHUNK = 5000
NRANGE = NUM_NODES // CHUNK
CPAD = 5120
DUMP = 5100
C_T = 312
C_TAIL = CHUNK - NS * C_T


def _sc_stage1(pk_hbm, node_emb, zero_hbm, one_hbm,
               esum_out, ecnt_out,
               pkc, nrow, erow, nrow2, erow2, rows, rows2, ones, acc,
               sem_a, sem_b):
    c = lax.axis_index("c")
    s = lax.axis_index("s")
    w = c * NS + s
    pltpu.sync_copy(one_hbm.at[pl.ds(0, B1)], ones)

    def zero_acc():
        for k in range(5):
            pltpu.sync_copy(zero_hbm, acc.at[pl.ds(s * 640 + k * B, B)])

    def dump_acc(dst):
        pltpu.sync_copy(acc.at[pl.ds(s * E_T, E_T)],
                        dst.at[c, pl.ds(s * E_T, E_T)])

        @pl.when(s == 0)
        def _():
            pltpu.sync_copy(acc.at[pl.ds(NS * E_T, E_TAIL)],
                            dst.at[c, pl.ds(NS * E_T, E_TAIL)])

    def sweep(phase):
        def chunk_loop(cc, _):
            pltpu.sync_copy(pk_hbm.at[pl.ds(w * S1P + cc * CH, CH)], pkc)

            def blk(j, _):
                for k in range(4):
                    pv = pkc[pl.ds(j * B1 + k * 16, 16)]
                    if phase == 0:
                        nrow[pl.ds(k * 16, 16)] = jnp.minimum(pv >> EBITS,
                                                              NUM_NODES - 1)
                    erow[pl.ds(k * 16, 16)] = pv & EMASK
                if phase == 0:
                    pltpu.sync_copy(node_emb.at[nrow], rows)
                    pltpu.sync_copy(rows, acc.at[erow], add=True)
                else:
                    pltpu.sync_copy(ones, acc.at[erow], add=True)
                return 0

            lax.fori_loop(0, CH // B1, blk, 0)
            return 0

        lax.fori_loop(0, S1P // CH, chunk_loop, 0)
        plsc.subcore_barrier()
        dump_acc(esum_out if phase == 0 else ecnt_out)
        plsc.subcore_barrier()

    zero_acc()
    plsc.subcore_barrier()
    sweep(0)
    sweep(1)


def _lane_gather(x, idx):
    dn = lax.GatherDimensionNumbers(offset_dims=(), collapsed_slice_dims=(0,),
                                    start_index_map=(0,))
    return lax.gather(x, idx[:, None], dn, (1,),
                      mode=lax.GatherScatterMode.PROMISE_IN_BOUNDS)


def _scalar(v):
    return lax.squeeze(lax.slice(v, (0,), (1,)), (0,))


def _sc_stage2(pk_hbm, ectx_hbm, zero_hbm, one_hbm,
               nsum_out, ncnt_out,
               pkc, comp, rows_a, rows_b, ones, sidx, sidx8, gidx, gidx2,
               chunk, sem_a, sem_b):
    c = lax.axis_index("c")
    s = lax.axis_index("s")
    iota = lax.iota(jnp.int32, 16)
    pltpu.sync_copy(one_hbm, ones)

    def zero_chunk():
        for off, n in ((0, B), (B, B), (2 * B, 64)):
            pltpu.sync_copy(zero_hbm.at[pl.ds(0, n)],
                            chunk.at[pl.ds(s * 320 + off, n)])

    def dump_chunk(dst, base):
        pltpu.sync_copy(chunk.at[pl.ds(s * C_T, C_T)],
                        dst.at[pl.ds(base + s * C_T, C_T)])

        @pl.when(s == 0)
        def _():
            pltpu.sync_copy(chunk.at[pl.ds(NS * C_T, C_TAIL)],
                            dst.at[pl.ds(base + NS * C_T, C_TAIL)])

    def one_pass(p, _):
        base = (NC * p + c) * CHUNK
        zero_chunk()
        plsc.subcore_barrier()

        def chunk_loop(cc, cur):
            pltpu.sync_copy(pk_hbm.at[pl.ds(s * S2P + cc * CH, CH)], pkc)

            def scan_blk(j, cur):
                basev = iota * 0 + base
                pbasev = basev * (1 << EBITS)
                for k in range(8):
                    pv = pkc[pl.ds(j * B + k * 16, 16)]
                    rel = (pv >> EBITS) - basev
                    oob = lax.shift_right_logical(rel | (CHUNK - 1 - rel), 31)
                    csum = 1 - oob
                    for sh in (1, 2, 4, 8):
                        down = _lane_gather(csum, jnp.maximum(iota - sh, 0))
                        csum = csum + jnp.where(iota >= sh, down, 0)
                    perm = iota * 0
                    tgt = iota + 1
                    for sh in (8, 4, 2, 1):
                        cand = perm + sh
                        cv = _lane_gather(csum, cand - 1)
                        perm = perm + sh * lax.shift_right_logical(cv - tgt,
                                                                   31)
                    comp[pl.ds(cur, 16)] = _lane_gather(pv - pbasev, perm)
                    cur = cur + lax.squeeze(lax.slice(csum, (15,), (16,)),
                                            (0,))
                return cur

            return lax.fori_loop(0, CH // B, scan_blk, cur)

        total = lax.fori_loop(0, S2P // CH, chunk_loop, jnp.int32(0))
        nfl = (total + B - 1) >> 7
        nf6 = (total + B2 - 1) >> 6

        def build_g(f, g):
            for k in range(4):
                fpk = iota + f * B2 + k * 16
                valid = fpk < iota * 0 + total
                pv = comp[pl.ds(f * B2 + k * 16, 16)]
                g[pl.ds(k * 16, 16)] = jnp.where(valid, pv & EMASK, 0)

        def build_s(f):
            for k in range(4):
                fpk = iota + f * B2 + k * 16
                valid = fpk < iota * 0 + total
                pv = comp[pl.ds(f * B2 + k * 16, 16)]
                sidx[pl.ds(k * 16, 16)] = jnp.where(valid, pv >> EBITS, DUMP)

        def build_idx(f, totalv, need_g):
            for k in range(8):
                fpk = iota + f * B + k * 16
                valid = fpk < totalv
                pv = comp[pl.ds(f * B + k * 16, 16)]
                sidx8[pl.ds(k * 16, 16)] = jnp.where(valid, pv >> EBITS, DUMP)

        def flush_b(f, _):
            build_idx(f, iota * 0 + total, False)
            pltpu.sync_copy(ones, chunk.at[sidx8], add=True)
            return 0

        @pl.when(nf6 > 0)
        def _():
            build_g(0, gidx)
            pltpu.async_copy(ectx_hbm.at[gidx], rows_a, sem_a).wait()

        def flush_pair(f2, _):
            f0 = 2 * f2
            f1 = f0 + 1

            @pl.when(f1 < nf6)
            def _():
                build_g(f1, gidx2)
                pltpu.async_copy(ectx_hbm.at[gidx2], rows_b, sem_b)
            build_s(f0)
            pltpu.sync_copy(rows_a, chunk.at[sidx], add=True)

            @pl.when(f1 < nf6)
            def _():
                pltpu.make_async_copy(ectx_hbm.at[gidx2], rows_b, sem_b).wait()

                @pl.when(f1 + 1 < nf6)
                def _():
                    build_g(f1 + 1, gidx)
                    pltpu.async_copy(ectx_hbm.at[gidx], rows_a, sem_a)
                build_s(f1)
                pltpu.sync_copy(rows_b, chunk.at[sidx], add=True)

                @pl.when(f1 + 1 < nf6)
                def _():
                    pltpu.make_async_copy(ectx_hbm.at[gidx], rows_a,
                                          sem_a).wait()
            return 0

        lax.fori_loop(0, (nf6 + 1) >> 1, flush_pair, 0)
        plsc.subcore_barrier()
        dump_chunk(nsum_out, base)
        plsc.subcore_barrier()
        lax.fori_loop(0, nfl, flush_b, 0)
        plsc.subcore_barrier()
        dump_chunk(ncnt_out, base)
        plsc.subcore_barrier()
        return 0

    lax.fori_loop(0, NRANGE // NC, one_pass, 0)


_MESH = plsc.VectorSubcoreMesh(core_axis_name="c", subcore_axis_name="s",
                               num_cores=NC, num_subcores=NS)

_stage1 = pl.kernel(
    _sc_stage1,
    out_type=(jax.ShapeDtypeStruct((NC, NUM_EDGES, D), jnp.float32),
              jax.ShapeDtypeStruct((NC, NUM_EDGES, D), jnp.float32)),
    mesh=_MESH,
    scratch_types=[
        pltpu.VMEM((CH,), jnp.int32),
        pltpu.VMEM((B1,), jnp.int32),
        pltpu.VMEM((B1,), jnp.int32),
        pltpu.VMEM((B1,), jnp.int32),
        pltpu.VMEM((B1,), jnp.int32),
        pltpu.VMEM((B1, D), jnp.float32),
        pltpu.VMEM((B1, D), jnp.float32),
        pltpu.VMEM((B1, D), jnp.float32),
        pltpu.VMEM_SHARED((EPAD, D), jnp.float32),
        pltpu.SemaphoreType.DMA,
        pltpu.SemaphoreType.DMA,
    ],
)

_stage2 = pl.kernel(
    _sc_stage2,
    out_type=(jax.ShapeDtypeStruct((NUM_NODES, D), jnp.float32),
              jax.ShapeDtypeStruct((NUM_NODES, D), jnp.float32)),
    mesh=_MESH,
    scratch_types=[
        pltpu.VMEM((CH,), jnp.int32),
        pltpu.VMEM((S2P,), jnp.int32),
        pltpu.VMEM((B2, D), jnp.float32),
        pltpu.VMEM((B2, D), jnp.float32),
        pltpu.VMEM((B, D), jnp.float32),
        pltpu.VMEM((B2,), jnp.int32),
        pltpu.VMEM((B,), jnp.int32),
        pltpu.VMEM((B2,), jnp.int32),
        pltpu.VMEM((B2,), jnp.int32),
        pltpu.VMEM_SHARED((CPAD, D), jnp.float32),
        pltpu.SemaphoreType.DMA,
        pltpu.SemaphoreType.DMA,
    ],
)

_BE = 1000
_BN = 2000


def _tc_edge(p_ref, c_ref, emb_ref, w_ref, b_ref, ctx_ref, out_ref):
    esum = p_ref[0] + p_ref[1]
    cnt = (c_ref[0][:, 0:1] - p_ref[0][:, 0:1]) + (c_ref[1][:, 0:1]
                                                   - p_ref[1][:, 0:1])
    mean = esum / jnp.maximum(cnt, 1.0)
    ctx = jnp.where(cnt > 0.5, mean, emb_ref[...])
    ctx = lax.dot_general(ctx, w_ref[...], (((1,), (1,)), ((), ())),
                          preferred_element_type=jnp.float32)
    ctx = ctx + b_ref[...]
    ctx_ref[...] = ctx
    out_ref[...] = emb_ref[...] + ctx


def _tc_node(sum_ref, cnt_ref, emb_ref, w_ref, b_ref, out_ref):
    total = cnt_ref[:, 0:1] - sum_ref[:, 0:1] + 1.0
    ctx = lax.dot_general(sum_ref[...] / total, w_ref[...],
                          (((1,), (1,)), ((), ())),
                          preferred_element_type=jnp.float32) + b_ref[...]
    out_ref[...] = emb_ref[...] + ctx


def _edge_tc(esum_p, ecnt_p, edge_emb, W_e, b_e2):
    return pl.pallas_call(
        _tc_edge,
        grid=(NUM_EDGES // _BE,),
        in_specs=[
            pl.BlockSpec((NC, _BE, D), lambda i: (0, i, 0)),
            pl.BlockSpec((NC, _BE, D), lambda i: (0, i, 0)),
            pl.BlockSpec((_BE, D), lambda i: (i, 0)),
            pl.BlockSpec((D, D), lambda i: (0, 0)),
            pl.BlockSpec((1, D), lambda i: (0, 0)),
        ],
        out_specs=[pl.BlockSpec((_BE, D), lambda i: (i, 0)),
                   pl.BlockSpec((_BE, D), lambda i: (i, 0))],
        out_shape=[jax.ShapeDtypeStruct((NUM_EDGES, D), jnp.float32),
                   jax.ShapeDtypeStruct((NUM_EDGES, D), jnp.float32)],
    )(esum_p, ecnt_p, edge_emb, W_e, b_e2)


def _node_tc(nsum, ncnt, node_emb, W_v, b_v2):
    return pl.pallas_call(
        _tc_node,
        grid=(NUM_NODES // _BN,),
        in_specs=[
            pl.BlockSpec((_BN, D), lambda i: (i, 0)),
            pl.BlockSpec((_BN, D), lambda i: (i, 0)),
            pl.BlockSpec((_BN, D), lambda i: (i, 0)),
            pl.BlockSpec((D, D), lambda i: (0, 0)),
            pl.BlockSpec((1, D), lambda i: (0, 0)),
        ],
        out_specs=pl.BlockSpec((_BN, D), lambda i: (i, 0)),
        out_shape=jax.ShapeDtypeStruct((NUM_NODES, D), jnp.float32),
    )(nsum, ncnt, node_emb, W_v, b_v2)


def kernel(node_ids, edge_ids, node_emb, edge_emb, W_e, b_e, W_v, b_v):
    packed = node_ids.astype(jnp.int32) * (1 << EBITS) + edge_ids.astype(
        jnp.int32)
    packed = jnp.pad(packed, (0, NPAD - NNZ), constant_values=PAD_PACK)
    zero_hbm = jnp.zeros((B, D), jnp.float32)
    one_hbm = jnp.ones((B, D), jnp.float32)
    esum_p, ecnt_p = _stage1(packed, node_emb, zero_hbm, one_hbm)
    edge_ctx, edge_out = _edge_tc(esum_p, ecnt_p, edge_emb, W_e,
                                  b_e.reshape(1, D))
    nsum, ncnt = _stage2(packed, edge_ctx, zero_hbm, one_hbm)
    node_out = _node_tc(nsum, ncnt, node_emb, W_v, b_v.reshape(1, D))
    return (node_out, edge_out)


# speedup vs baseline: 2.1739x; 1.0313x over previous
import jax
import jax.numpy as jnp
from jax import lax
from jax.experimental import pallas as pl
from jax.experimental.pallas import tpu as pltpu
from jax.experimental.pallas import tpu_sc as plsc

NUM_NODES = 100000
NUM_EDGES = 10000
D = 128
NNZ = 320000

NC, NS = 2, 16
B = 128
B1 = 64
B2 = 64
PAD_BLKS = 2560
NPAD = PAD_BLKS * B
S1P = NPAD // (NC * NS)
S2P = NPAD // NS
CH = 2048
EBITS = 14
EMASK = (1 << EBITS) - 1
PAD_NID = (1 << 17) - 1
PAD_EID = 10100
PAD_PACK = PAD_NID * (1 << EBITS) + PAD_EID
EPAD = 10240
E_T = 624
E_TAIL = NUM_EDGES - NS * E_T
CHUNK = 5000
NRANGE = NUM_NODES // CHUNK
CPAD = 5120
DUMP = 5100
C_T = 312
C_TAIL = CHUNK - NS * C_T


def _sc_stage1(pk_hbm, node_emb, zero_hbm, one_hbm,
               esum_out, ecnt_out,
               pkc, nrow, erow, rows, ones, acc):
    c = lax.axis_index("c")
    s = lax.axis_index("s")
    w = c * NS + s
    pltpu.sync_copy(one_hbm.at[pl.ds(0, B1)], ones)

    def zero_acc():
        for k in range(5):
            pltpu.sync_copy(zero_hbm, acc.at[pl.ds(s * 640 + k * B, B)])

    def dump_acc(dst):
        pltpu.sync_copy(acc.at[pl.ds(s * E_T, E_T)],
                        dst.at[c, pl.ds(s * E_T, E_T)])

        @pl.when(s == 0)
        def _():
            pltpu.sync_copy(acc.at[pl.ds(NS * E_T, E_TAIL)],
                            dst.at[c, pl.ds(NS * E_T, E_TAIL)])

    def sweep(phase):
        def chunk_loop(cc, _):
            pltpu.sync_copy(pk_hbm.at[pl.ds(w * S1P + cc * CH, CH)], pkc)

            def blk(j, _):
                for k in range(4):
                    pv = pkc[pl.ds(j * B1 + k * 16, 16)]
                    if phase == 0:
                        nrow[pl.ds(k * 16, 16)] = jnp.minimum(pv >> EBITS,
                                                              NUM_NODES - 1)
                    erow[pl.ds(k * 16, 16)] = pv & EMASK
                if phase == 0:
                    pltpu.sync_copy(node_emb.at[nrow], rows)
                    pltpu.sync_copy(rows, acc.at[erow], add=True)
                else:
                    pltpu.sync_copy(ones, acc.at[erow], add=True)
                return 0

            lax.fori_loop(0, CH // B1, blk, 0)
            return 0

        lax.fori_loop(0, S1P // CH, chunk_loop, 0)
        plsc.subcore_barrier()
        dump_acc(esum_out if phase == 0 else ecnt_out)
        plsc.subcore_barrier()

    zero_acc()
    plsc.subcore_barrier()
    sweep(0)
    sweep(1)


def _lane_gather(x, idx):
    dn = lax.GatherDimensionNumbers(offset_dims=(), collapsed_slice_dims=(0,),
                                    start_index_map=(0,))
    return lax.gather(x, idx[:, None], dn, (1,),
                      mode=lax.GatherScatterMode.PROMISE_IN_BOUNDS)


def _scalar(v):
    return lax.squeeze(lax.slice(v, (0,), (1,)), (0,))


def _sc_stage2(pk_hbm, ectx_hbm, zero_hbm, one_hbm,
               nsum_out, ncnt_out,
               pk1d, comp, rows_a, rows_b, ones, sidx, gidx, gidx2,
               chunk, sem_a, sem_b):
    c = lax.axis_index("c")
    s = lax.axis_index("s")
    iota = lax.iota(jnp.int32, 16)
    pltpu.sync_copy(one_hbm.at[pl.ds(0, B2)], ones)
    pltpu.sync_copy(pk_hbm.at[pl.ds(s * S2P, S2P)], pk1d)

    def zero_chunk():
        for off, n in ((0, B), (B, B), (2 * B, 64)):
            pltpu.sync_copy(zero_hbm.at[pl.ds(0, n)],
                            chunk.at[pl.ds(s * 320 + off, n)])

    def dump_chunk(dst, base):
        pltpu.sync_copy(chunk.at[pl.ds(s * C_T, C_T)],
                        dst.at[pl.ds(base + s * C_T, C_T)])

        @pl.when(s == 0)
        def _():
            pltpu.sync_copy(chunk.at[pl.ds(NS * C_T, C_TAIL)],
                            dst.at[pl.ds(base + NS * C_T, C_TAIL)])

    def one_pass(p, _):
        base = (NC * p + c) * CHUNK
        zero_chunk()
        plsc.subcore_barrier()

        def scan_blk(j, cur):
                basev = iota * 0 + base
                pbasev = basev * (1 << EBITS)
                for k in range(8):
                    pv = pk1d[pl.ds(j * B + k * 16, 16)]
                    rel = (pv >> EBITS) - basev
                    oob = lax.shift_right_logical(rel | (CHUNK - 1 - rel), 31)
                    csum = 1 - oob
                    for sh in (1, 2, 4, 8):
                        down = _lane_gather(csum, jnp.maximum(iota - sh, 0))
                        csum = csum + jnp.where(iota >= sh, down, 0)
                    perm = iota * 0
                    tgt = iota + 1
                    for sh in (8, 4, 2, 1):
                        cand = perm + sh
                        cv = _lane_gather(csum, cand - 1)
                        perm = perm + sh * lax.shift_right_logical(cv - tgt,
                                                                   31)
                    comp[pl.ds(cur, 16)] = _lane_gather(pv - pbasev, perm)
                    cur = cur + lax.squeeze(lax.slice(csum, (15,), (16,)),
                                            (0,))
                return cur

        total = lax.fori_loop(0, S2P // B, scan_blk, jnp.int32(0))
        nf6 = (total + B2 - 1) >> 6

        def build_g(f, g):
            for k in range(4):
                fpk = iota + f * B2 + k * 16
                valid = fpk < iota * 0 + total
                pv = comp[pl.ds(f * B2 + k * 16, 16)]
                g[pl.ds(k * 16, 16)] = jnp.where(valid, pv & EMASK, 0)

        def build_s(f):
            for k in range(4):
                fpk = iota + f * B2 + k * 16
                valid = fpk < iota * 0 + total
                pv = comp[pl.ds(f * B2 + k * 16, 16)]
                sidx[pl.ds(k * 16, 16)] = jnp.where(valid, pv >> EBITS, DUMP)

        def flush_b(f, _):
            build_s(f)
            pltpu.sync_copy(ones, chunk.at[sidx], add=True)
            return 0

        @pl.when(nf6 > 0)
        def _():
            build_g(0, gidx)
            pltpu.async_copy(ectx_hbm.at[gidx], rows_a, sem_a).wait()

        def flush_pair(f2, _):
            f0 = 2 * f2
            f1 = f0 + 1

            @pl.when(f1 < nf6)
            def _():
                build_g(f1, gidx2)
                pltpu.async_copy(ectx_hbm.at[gidx2], rows_b, sem_b)
            build_s(f0)
            pltpu.sync_copy(rows_a, chunk.at[sidx], add=True)

            @pl.when(f1 < nf6)
            def _():
                pltpu.make_async_copy(ectx_hbm.at[gidx2], rows_b, sem_b).wait()

                @pl.when(f1 + 1 < nf6)
                def _():
                    build_g(f1 + 1, gidx)
                    pltpu.async_copy(ectx_hbm.at[gidx], rows_a, sem_a)
                build_s(f1)
                pltpu.sync_copy(rows_b, chunk.at[sidx], add=True)

                @pl.when(f1 + 1 < nf6)
                def _():
                    pltpu.make_async_copy(ectx_hbm.at[gidx], rows_a,
                                          sem_a).wait()
            return 0

        lax.fori_loop(0, (nf6 + 1) >> 1, flush_pair, 0)
        plsc.subcore_barrier()
        dump_chunk(nsum_out, base)
        plsc.subcore_barrier()
        lax.fori_loop(0, nf6, flush_b, 0)
        plsc.subcore_barrier()
        dump_chunk(ncnt_out, base)
        plsc.subcore_barrier()
        return 0

    lax.fori_loop(0, NRANGE // NC, one_pass, 0)


_MESH = plsc.VectorSubcoreMesh(core_axis_name="c", subcore_axis_name="s",
                               num_cores=NC, num_subcores=NS)

_stage1 = pl.kernel(
    _sc_stage1,
    out_type=(jax.ShapeDtypeStruct((NC, NUM_EDGES, D), jnp.float32),
              jax.ShapeDtypeStruct((NC, NUM_EDGES, D), jnp.float32)),
    mesh=_MESH,
    scratch_types=[
        pltpu.VMEM((CH,), jnp.int32),
        pltpu.VMEM((B1,), jnp.int32),
        pltpu.VMEM((B1,), jnp.int32),
        pltpu.VMEM((B1, D), jnp.float32),
        pltpu.VMEM((B1, D), jnp.float32),
        pltpu.VMEM_SHARED((EPAD, D), jnp.float32),
    ],
)

_stage2 = pl.kernel(
    _sc_stage2,
    out_type=(jax.ShapeDtypeStruct((NUM_NODES, D), jnp.float32),
              jax.ShapeDtypeStruct((NUM_NODES, D), jnp.float32)),
    mesh=_MESH,
    scratch_types=[
        pltpu.VMEM((S2P,), jnp.int32),
        pltpu.VMEM((S2P,), jnp.int32),
        pltpu.VMEM((B2, D), jnp.float32),
        pltpu.VMEM((B2, D), jnp.float32),
        pltpu.VMEM((B2, D), jnp.float32),
        pltpu.VMEM((B2,), jnp.int32),
        pltpu.VMEM((B2,), jnp.int32),
        pltpu.VMEM((B2,), jnp.int32),
        pltpu.VMEM_SHARED((CPAD, D), jnp.float32),
        pltpu.SemaphoreType.DMA,
        pltpu.SemaphoreType.DMA,
    ],
)

_BE = 1000
_BN = 2000


def _tc_edge(p_ref, c_ref, emb_ref, w_ref, b_ref, ctx_ref, out_ref):
    esum = p_ref[0] + p_ref[1]
    cnt = (c_ref[0][:, 0:1] - p_ref[0][:, 0:1]) + (c_ref[1][:, 0:1]
                                                   - p_ref[1][:, 0:1])
    mean = esum / jnp.maximum(cnt, 1.0)
    ctx = jnp.where(cnt > 0.5, mean, emb_ref[...])
    ctx = lax.dot_general(ctx, w_ref[...], (((1,), (1,)), ((), ())),
                          preferred_element_type=jnp.float32)
    ctx = ctx + b_ref[...]
    ctx_ref[...] = ctx
    out_ref[...] = emb_ref[...] + ctx


def _tc_node(sum_ref, cnt_ref, emb_ref, w_ref, b_ref, out_ref):
    total = cnt_ref[:, 0:1] - sum_ref[:, 0:1] + 1.0
    ctx = lax.dot_general(sum_ref[...] / total, w_ref[...],
                          (((1,), (1,)), ((), ())),
                          preferred_element_type=jnp.float32) + b_ref[...]
    out_ref[...] = emb_ref[...] + ctx


def _edge_tc(esum_p, ecnt_p, edge_emb, W_e, b_e2):
    return pl.pallas_call(
        _tc_edge,
        grid=(NUM_EDGES // _BE,),
        in_specs=[
            pl.BlockSpec((NC, _BE, D), lambda i: (0, i, 0)),
            pl.BlockSpec((NC, _BE, D), lambda i: (0, i, 0)),
            pl.BlockSpec((_BE, D), lambda i: (i, 0)),
            pl.BlockSpec((D, D), lambda i: (0, 0)),
            pl.BlockSpec((1, D), lambda i: (0, 0)),
        ],
        out_specs=[pl.BlockSpec((_BE, D), lambda i: (i, 0)),
                   pl.BlockSpec((_BE, D), lambda i: (i, 0))],
        out_shape=[jax.ShapeDtypeStruct((NUM_EDGES, D), jnp.float32),
                   jax.ShapeDtypeStruct((NUM_EDGES, D), jnp.float32)],
    )(esum_p, ecnt_p, edge_emb, W_e, b_e2)


def _node_tc(nsum, ncnt, node_emb, W_v, b_v2):
    return pl.pallas_call(
        _tc_node,
        grid=(NUM_NODES // _BN,),
        in_specs=[
            pl.BlockSpec((_BN, D), lambda i: (i, 0)),
            pl.BlockSpec((_BN, D), lambda i: (i, 0)),
            pl.BlockSpec((_BN, D), lambda i: (i, 0)),
            pl.BlockSpec((D, D), lambda i: (0, 0)),
            pl.BlockSpec((1, D), lambda i: (0, 0)),
        ],
        out_specs=pl.BlockSpec((_BN, D), lambda i: (i, 0)),
        out_shape=jax.ShapeDtypeStruct((NUM_NODES, D), jnp.float32),
    )(nsum, ncnt, node_emb, W_v, b_v2)


def kernel(node_ids, edge_ids, node_emb, edge_emb, W_e, b_e, W_v, b_v):
    packed = node_ids.astype(jnp.int32) * (1 << EBITS) + edge_ids.astype(
        jnp.int32)
    packed = jnp.pad(packed, (0, NPAD - NNZ), constant_values=PAD_PACK)
    zero_hbm = jnp.zeros((B, D), jnp.float32)
    one_hbm = jnp.ones((B, D), jnp.float32)
    esum_p, ecnt_p = _stage1(packed, node_emb, zero_hbm, one_hbm)
    edge_ctx, edge_out = _edge_tc(esum_p, ecnt_p, edge_emb, W_e,
                                  b_e.reshape(1, D))
    nsum, ncnt = _stage2(packed, edge_ctx, zero_hbm, one_hbm)
    node_out = _node_tc(nsum, ncnt, node_emb, W_v, b_v.reshape(1, D))
    return (node_out, edge_out)
